# baseline (speedup 1.0000x reference)
import functools

import jax
import jax.numpy as jnp
from jax import lax
from jax.experimental import pallas as pl
from jax.experimental.pallas import tpu as pltpu
from jax.experimental.pallas import tpu_sc as plsc

_BATCH = 4096
_FIELDS = 26
_K = 32
_NC = 2
_NS = 16
_V = 1000001
_VP = 1000064
_VPAD = 1000064
_NST = 2
_KH = _K // _NST
_KPC = _KH // _NC
_BPT = _BATCH // _NS
_ROWS_PT = _FIELDS * _BPT // 128
_LCH = 62496
_LCH_LAST = _LCH + (_VP - _NS * _LCH)
_DW = 76928


_NCI = _VP // _DW
_LSUB = _DW // _NS


def _detile_block(i_ref, o_ref):
    for r in range(8):
        o_ref[pl.ds(r * _DW, _DW)] = i_ref[r, :]


def _make_detile(h):
    return pl.pallas_call(
        _detile_block,
        grid=(_KH // 8, _NCI),
        in_specs=[pl.BlockSpec((8, _DW), lambda g, ci: (h * (_KH // 8) + g, ci))],
        out_specs=pl.BlockSpec((8 * _DW,), lambda g, ci: (g * _NCI + ci,)),
        out_shape=jax.ShapeDtypeStruct((_KH * _VP,), jnp.float32),
    )


@functools.partial(
    pl.kernel,
    out_type=jax.ShapeDtypeStruct((_FIELDS, _KH, _BATCH), jnp.float32),
    mesh=plsc.VectorSubcoreMesh(core_axis_name="c", subcore_axis_name="s"),
    scratch_types=[
        pltpu.VMEM_SHARED((_VPAD,), jnp.float32),
        pltpu.VMEM((_ROWS_PT, 128), jnp.int32),
        pltpu.VMEM((_ROWS_PT, 128), jnp.float32),
        pltpu.VMEM((_ROWS_PT, 128), jnp.float32),
        pltpu.VMEM((_ROWS_PT, 128), jnp.float32),
        pltpu.SemaphoreType.DMA,
        pltpu.SemaphoreType.DMA,
        pltpu.SemaphoreType.DMA,
    ],
    compiler_params=pltpu.CompilerParams(use_tc_tiling_on_sc=False),
)
def _plane_gather(idx_hbm, val_hbm, table_hbm, out_hbm,
                  plane, idx_v, val_v, g_v, g2_v, sem_l, sem_g, sem_o):
    c = lax.axis_index("c")
    s = lax.axis_index("s")
    k0 = c * _KPC

    pltpu.sync_copy(idx_hbm.at[s], idx_v)
    pltpu.sync_copy(val_hbm.at[s], val_v)

    def load_plane(k):
        g = k // 8
        r = lax.rem(k, 8)
        for ci in range(_NCI):
            src = pl.multiple_of(
                ((g * _NCI + ci) * 8 + r) * _DW + s * _LSUB, 8
            )
            pltpu.async_copy(
                table_hbm.at[pl.ds(src, _LSUB)],
                plane.at[pl.ds(ci * _DW + s * _LSUB, _LSUB)],
                sem_l,
            )

    def wait_plane():
        for _ in range(_NCI):
            pltpu.make_async_copy(
                table_hbm.at[pl.ds(0, _LSUB)],
                plane.at[pl.ds(0, _LSUB)],
                sem_l,
            ).wait()

        plsc.subcore_barrier()

    load_plane(k0)
    wait_plane()

    def process(k, g, not_last):
        descs = [
            pltpu.async_copy(plane.at[idx_v.at[r]], g.at[r], sem_g)
            for r in range(_ROWS_PT)
        ]
        for d in descs:
            d.wait()

        plsc.subcore_barrier()

        @pl.when(not_last)
        def _():
            load_plane(k + 1)

        def mul_row(r, carry):
            for cc in range(8):
                sl = pl.ds(cc * 16, 16)
                g[r, sl] = g[r, sl] * val_v[r, sl]
            return carry

        lax.fori_loop(0, _ROWS_PT, mul_row, 0)

        pltpu.async_copy(
            g.at[pl.ds(0, _FIELDS)],
            out_hbm.at[:, k, pl.ds(s * _BPT, 128)],
            sem_o,
        )
        pltpu.async_copy(
            g.at[pl.ds(_FIELDS, _FIELDS)],
            out_hbm.at[:, k, pl.ds(s * _BPT + 128, 128)],
            sem_o,
        )

        @pl.when(not_last)
        def _():
            wait_plane()

    def drain_stores(g):
        pltpu.make_async_copy(
            g.at[pl.ds(0, _FIELDS)],
            out_hbm.at[:, 0, pl.ds(s * _BPT, 128)],
            sem_o,
        ).wait()
        pltpu.make_async_copy(
            g.at[pl.ds(_FIELDS, _FIELDS)],
            out_hbm.at[:, 0, pl.ds(s * _BPT + 128, 128)],
            sem_o,
        ).wait()

    def body(j, carry):
        k = k0 + 2 * j
        not_first = j > 0

        @pl.when(not_first)
        def _():
            drain_stores(g_v)

        process(k, g_v, j >= 0)

        @pl.when(not_first)
        def _():
            drain_stores(g2_v)

        process(k + 1, g2_v, j < _KPC // 2 - 1)
        return carry

    lax.fori_loop(0, _KPC // 2, body, 0)
    drain_stores(g_v)
    drain_stores(g2_v)


def kernel(nonzero_index, nonzero_value, table):
    idx_t = nonzero_index.astype(jnp.int32).T
    val_t = nonzero_value.T
    tt = table.T

    def slab(x):
        return (
            x.reshape(_FIELDS, _BATCH // 128, 128)
            .transpose(1, 0, 2)
            .reshape(_NS, _ROWS_PT, 128)
        )

    idx_s, val_s = slab(idx_t), slab(val_t)
    outs = [
        _plane_gather(idx_s, val_s, _make_detile(h)(tt))
        for h in range(_NST)
    ]
    out_t = jnp.concatenate(outs, axis=1)
    return out_t.transpose(2, 0, 1)


# speedup vs baseline: 1.0060x; 1.0060x over previous
import functools

import jax
import jax.numpy as jnp
from jax import lax
from jax.experimental import pallas as pl
from jax.experimental.pallas import tpu as pltpu
from jax.experimental.pallas import tpu_sc as plsc

_BATCH = 4096
_FIELDS = 26
_K = 32
_NC = 2
_NS = 16
_V = 1000001
_VP = 1000064
_VPAD = 1000064
_NST = 2
_KH = _K // _NST
_KPC = _KH // _NC
_BPT = _BATCH // _NS
_ROWS_PT = _FIELDS * _BPT // 128
_LCH = 62496
_LCH_LAST = _LCH + (_VP - _NS * _LCH)
_DW = 76928


_NCI = _VP // _DW
_LSUB = _DW // _NS


def _detile_block(i_ref, o_ref):
    for r in range(8):
        o_ref[pl.ds(r * _DW, _DW)] = i_ref[r, :]


def _make_detile(h):
    return pl.pallas_call(
        _detile_block,
        grid=(_KH // 8, _NCI),
        in_specs=[pl.BlockSpec((8, _DW), lambda g, ci: (h * (_KH // 8) + g, ci))],
        out_specs=pl.BlockSpec((8 * _DW,), lambda g, ci: (g * _NCI + ci,)),
        out_shape=jax.ShapeDtypeStruct((_KH * _VP,), jnp.float32),
    )


@functools.partial(
    pl.kernel,
    out_type=jax.ShapeDtypeStruct((_FIELDS, _KH, _BATCH), jnp.float32),
    mesh=plsc.VectorSubcoreMesh(core_axis_name="c", subcore_axis_name="s"),
    scratch_types=[
        pltpu.VMEM_SHARED((_VPAD,), jnp.float32),
        pltpu.VMEM((_ROWS_PT, 128), jnp.int32),
        pltpu.VMEM((_ROWS_PT, 128), jnp.float32),
        pltpu.VMEM((_ROWS_PT, 128), jnp.float32),
        pltpu.SemaphoreType.DMA,
        pltpu.SemaphoreType.DMA,
        pltpu.SemaphoreType.DMA,
    ],
    compiler_params=pltpu.CompilerParams(use_tc_tiling_on_sc=False),
)
def _plane_gather(idx_hbm, val_hbm, table_hbm, out_hbm,
                  plane, idx_v, val_v, g_v, sem_l, sem_g, sem_o):
    c = lax.axis_index("c")
    s = lax.axis_index("s")
    k0 = c * _KPC

    pltpu.sync_copy(idx_hbm.at[s], idx_v)
    pltpu.sync_copy(val_hbm.at[s], val_v)

    def load_plane(k):
        g = k // 8
        r = lax.rem(k, 8)
        for ci in range(_NCI):
            src = pl.multiple_of(
                ((g * _NCI + ci) * 8 + r) * _DW + s * _LSUB, 8
            )
            pltpu.async_copy(
                table_hbm.at[pl.ds(src, _LSUB)],
                plane.at[pl.ds(ci * _DW + s * _LSUB, _LSUB)],
                sem_l,
            )

    def wait_plane():
        for _ in range(_NCI):
            pltpu.make_async_copy(
                table_hbm.at[pl.ds(0, _LSUB)],
                plane.at[pl.ds(0, _LSUB)],
                sem_l,
            ).wait()

        plsc.subcore_barrier()

    load_plane(k0)
    wait_plane()

    def body(i, carry):
        k = k0 + i

        descs = [
            pltpu.async_copy(plane.at[idx_v.at[r]], g_v.at[r], sem_g)
            for r in range(_ROWS_PT)
        ]
        for d in descs:
            d.wait()

        plsc.subcore_barrier()

        @pl.when(i < _KPC - 1)
        def _():
            load_plane(k + 1)

        def mul_row(r, carry):
            for cc in range(8):
                sl = pl.ds(cc * 16, 16)
                g_v[r, sl] = g_v[r, sl] * val_v[r, sl]
            return carry

        lax.fori_loop(0, _ROWS_PT, mul_row, 0)

        d0 = pltpu.async_copy(
            g_v.at[pl.ds(0, _FIELDS)],
            out_hbm.at[:, k, pl.ds(s * _BPT, 128)],
            sem_o,
        )
        d1 = pltpu.async_copy(
            g_v.at[pl.ds(_FIELDS, _FIELDS)],
            out_hbm.at[:, k, pl.ds(s * _BPT + 128, 128)],
            sem_o,
        )
        d0.wait()
        d1.wait()

        @pl.when(i < _KPC - 1)
        def _():
            wait_plane()

        return carry

    lax.fori_loop(0, _KPC, body, 0)


def kernel(nonzero_index, nonzero_value, table):
    idx_t = nonzero_index.astype(jnp.int32).T
    val_t = nonzero_value.T
    tt = table.T

    def slab(x):
        return (
            x.reshape(_FIELDS, _BATCH // 128, 128)
            .transpose(1, 0, 2)
            .reshape(_NS, _ROWS_PT, 128)
        )

    idx_s, val_s = slab(idx_t), slab(val_t)
    outs = [
        _plane_gather(idx_s, val_s, _make_detile(h)(tt))
        for h in range(_NST)
    ]
    out_t = jnp.concatenate(outs, axis=1)
    return out_t.transpose(2, 0, 1)
